# baseline (speedup 1.0000x reference)
import functools

import jax
import jax.numpy as jnp
from jax import lax
from jax.experimental import pallas as pl
from jax.experimental.pallas import tpu as pltpu
from jax.experimental.pallas import tpu_sc as plsc

N = 16384
DIM = 32
ROWS = 1000000
PACK = 128 // DIM
QROWS = ROWS // PACK
REG_USER = 0.0025
REG_POS_ITEM = 0.0025
REG_NEG_ITEM = 0.00025
REG_BIAS = 0.001

_INFO = plsc.get_sparse_core_info()
_NC = _INFO.num_cores
_NS = _INFO.num_subcores
_NW = _NC * _NS
_L = 16

_CH = 512
_NFULL = (ROWS - 64) // _CH
_TAIL0 = _NFULL * _CH

_BPW = N // _NW
_CHUNK = 256
_NCHUNK = _BPW // _CHUNK


def _repack_chunk(cbuf, obuf, iota, width):
    for g in range(width // _L):
        r32 = (g * _L + iota) * DIM
        for d in range(DIM):
            t = r32 + d
            plsc.store_scatter(
                obuf,
                [jnp.right_shift(t, 7), jnp.bitwise_and(t, 127)],
                cbuf[d, pl.ds(g * _L, _L)],
            )


def _k0_body(w_hbm, h_hbm, wo_hbm, ho_hbm,
             cw_v, ch_v, ow_v, oh_v, ct_v, isem, osem):
    wid = lax.axis_index("s") * _NC + lax.axis_index("c")
    lo = jnp.right_shift(wid * _NFULL, 5)
    hi = jnp.right_shift((wid + 1) * _NFULL, 5)
    iota = lax.iota(jnp.int32, _L)

    def chunk(c, _):
        c0 = c * _CH
        q0 = c * (_CH // PACK)
        ci = pltpu.async_copy(w_hbm.at[:, pl.ds(c0, _CH)], cw_v, isem)
        ch = pltpu.async_copy(h_hbm.at[:, pl.ds(c0, _CH)], ch_v, isem)

        @pl.when(c > lo)
        def _drain():
            pltpu.make_async_copy(ow_v, wo_hbm.at[pl.ds(0, _CH // PACK)], osem).wait()
            pltpu.make_async_copy(oh_v, ho_hbm.at[pl.ds(0, _CH // PACK)], osem).wait()

        ci.wait()
        _repack_chunk(cw_v, ow_v, iota, _CH)
        pltpu.async_copy(ow_v, wo_hbm.at[pl.ds(q0, _CH // PACK)], osem)
        ch.wait()
        _repack_chunk(ch_v, oh_v, iota, _CH)
        pltpu.async_copy(oh_v, ho_hbm.at[pl.ds(q0, _CH // PACK)], osem)
        return 0

    lax.fori_loop(lo, hi, chunk, 0)
    pltpu.make_async_copy(ow_v, wo_hbm.at[pl.ds(0, _CH // PACK)], osem).wait()
    pltpu.make_async_copy(oh_v, ho_hbm.at[pl.ds(0, _CH // PACK)], osem).wait()

    @pl.when(wid == 0)
    def _tail():
        tw = 64
        qt = _TAIL0 // PACK
        pltpu.async_copy(w_hbm.at[:, pl.ds(_TAIL0, tw)], ct_v, isem).wait()
        _repack_chunk(ct_v, ow_v, iota, tw)
        pltpu.sync_copy(ow_v.at[pl.ds(0, tw // PACK)], wo_hbm.at[pl.ds(qt, tw // PACK)])
        pltpu.async_copy(h_hbm.at[:, pl.ds(_TAIL0, tw)], ct_v, isem).wait()
        _repack_chunk(ct_v, oh_v, iota, tw)
        pltpu.sync_copy(oh_v.at[pl.ds(0, tw // PACK)], ho_hbm.at[pl.ds(qt, tw // PACK)])


def _k1_body(u_hbm, i_hbm, j_hbm, w_hbm, h_hbm, b_hbm,
             x_hbm, swu_hbm, shi_hbm, shj_hbm, bs_hbm,
             u_v, i_v, j_v, uq_v, iq_v, jq_v, wu_v, hi_v, hj_v, bi_v, bj_v,
             x_v, swu_v, shi_v, shj_v, bs_v, sem):
    wid = lax.axis_index("s") * _NC + lax.axis_index("c")
    base = wid * _BPW

    pltpu.sync_copy(u_hbm.at[pl.ds(base, _BPW)], u_v)
    pltpu.sync_copy(i_hbm.at[pl.ds(base, _BPW)], i_v)
    pltpu.sync_copy(j_hbm.at[pl.ds(base, _BPW)], j_v)

    for s in range(_BPW // _L):
        sl = pl.ds(s * _L, _L)
        uq_v[sl] = jnp.right_shift(u_v[sl], 2)
        iq_v[sl] = jnp.right_shift(i_v[sl], 2)
        jq_v[sl] = jnp.right_shift(j_v[sl], 2)

    cb1 = pltpu.async_copy(b_hbm.at[i_v], bi_v, sem)
    cb2 = pltpu.async_copy(b_hbm.at[j_v], bj_v, sem)

    iota = lax.iota(jnp.int32, _L)

    for ch in range(_NCHUNK):
        c0 = ch * _CHUNK
        c1 = pltpu.async_copy(w_hbm.at[uq_v.at[pl.ds(c0, _CHUNK)]], wu_v, sem)
        c2 = pltpu.async_copy(h_hbm.at[iq_v.at[pl.ds(c0, _CHUNK)]], hi_v, sem)
        c3 = pltpu.async_copy(h_hbm.at[jq_v.at[pl.ds(c0, _CHUNK)]], hj_v, sem)
        c1.wait()
        c2.wait()
        c3.wait()

        def block(blk, _):
            r0 = blk * _L
            rows = r0 + iota
            g0 = c0 + r0

            def soff(v):
                return jnp.left_shift(jnp.bitwise_and(v, PACK - 1), 5)

            uoff = soff(u_v[pl.ds(g0, _L)])
            ioff = soff(i_v[pl.ds(g0, _L)])
            joff = soff(j_v[pl.ds(g0, _L)])
            zero = jnp.zeros((_L,), jnp.float32)
            acc_ui = zero
            acc_uj = zero
            acc_wu = zero
            acc_hi = zero
            acc_hj = zero
            for d in range(DIM):
                cw = plsc.load_gather(wu_v, [rows, uoff + d])
                ci = plsc.load_gather(hi_v, [rows, ioff + d])
                cj = plsc.load_gather(hj_v, [rows, joff + d])
                acc_ui = acc_ui + cw * ci
                acc_uj = acc_uj + cw * cj
                acc_wu = acc_wu + cw * cw
                acc_hi = acc_hi + ci * ci
                acc_hj = acc_hj + cj * cj
            bi = bi_v[pl.ds(g0, _L)]
            bj = bj_v[pl.ds(g0, _L)]
            x_v[pl.ds(g0, _L)] = acc_ui - acc_uj + bi - bj
            swu_v[pl.ds(g0, _L)] = acc_wu
            shi_v[pl.ds(g0, _L)] = acc_hi
            shj_v[pl.ds(g0, _L)] = acc_hj
            bs_v[pl.ds(g0, _L)] = bi + bj
            return 0

        if ch == 0:
            cb1.wait()
            cb2.wait()
        lax.fori_loop(0, _CHUNK // _L, block, 0)

    pltpu.sync_copy(x_v, x_hbm.at[pl.ds(base, _BPW)])
    pltpu.sync_copy(swu_v, swu_hbm.at[pl.ds(base, _BPW)])
    pltpu.sync_copy(shi_v, shi_hbm.at[pl.ds(base, _BPW)])
    pltpu.sync_copy(shj_v, shj_hbm.at[pl.ds(base, _BPW)])
    pltpu.sync_copy(bs_v, bs_hbm.at[pl.ds(base, _BPW)])


@jax.jit
def _sc_partials(u, i, j, W, H, B):
    f32 = jnp.float32
    mesh = plsc.VectorSubcoreMesh(core_axis_name="c", subcore_axis_name="s")
    params = pltpu.CompilerParams(needs_layout_passes=False)

    Ww, Hw = pl.kernel(
        _k0_body,
        mesh=mesh,
        compiler_params=params,
        out_type=[jax.ShapeDtypeStruct((QROWS, PACK * DIM), f32)] * 2,
        scratch_types=[
            pltpu.VMEM((DIM, _CH), f32),
            pltpu.VMEM((DIM, _CH), f32),
            pltpu.VMEM((_CH // PACK, PACK * DIM), f32),
            pltpu.VMEM((_CH // PACK, PACK * DIM), f32),
            pltpu.VMEM((DIM, 64), f32),
            pltpu.SemaphoreType.DMA,
            pltpu.SemaphoreType.DMA,
        ],
    )(W.T, H.T)

    out = pl.kernel(
        _k1_body,
        mesh=mesh,
        compiler_params=params,
        out_type=[jax.ShapeDtypeStruct((N,), f32) for _ in range(5)],
        scratch_types=[
            pltpu.VMEM((_BPW,), jnp.int32),
            pltpu.VMEM((_BPW,), jnp.int32),
            pltpu.VMEM((_BPW,), jnp.int32),
            pltpu.VMEM((_BPW,), jnp.int32),
            pltpu.VMEM((_BPW,), jnp.int32),
            pltpu.VMEM((_BPW,), jnp.int32),
            pltpu.VMEM((_CHUNK, PACK * DIM), f32),
            pltpu.VMEM((_CHUNK, PACK * DIM), f32),
            pltpu.VMEM((_CHUNK, PACK * DIM), f32),
            pltpu.VMEM((_BPW,), f32),
            pltpu.VMEM((_BPW,), f32),
            pltpu.VMEM((_BPW,), f32),
            pltpu.VMEM((_BPW,), f32),
            pltpu.VMEM((_BPW,), f32),
            pltpu.VMEM((_BPW,), f32),
            pltpu.VMEM((_BPW,), f32),
            pltpu.SemaphoreType.DMA,
        ],
    )(u, i, j, Ww, Hw, B)
    return out


def _tc_body(x_ref, swu_ref, shi_ref, shj_ref, bs_ref, out_ref):
    x = x_ref[...]
    lp = jnp.mean(-jnp.log(1.0 + jnp.exp(-x)))
    lp = lp - REG_USER * jnp.mean(jnp.sqrt(swu_ref[...]))
    lp = lp - REG_POS_ITEM * jnp.mean(jnp.sqrt(shi_ref[...]))
    lp = lp - REG_NEG_ITEM * jnp.mean(jnp.sqrt(shj_ref[...]))
    lp = lp - REG_BIAS * jnp.mean(bs_ref[...])
    out_ref[0, 0] = -lp


@jax.jit
def _tc_reduce(x, swu, shi, shj, bs):
    r = lambda a: a.reshape(128, 128)
    out = pl.pallas_call(
        _tc_body,
        out_shape=jax.ShapeDtypeStruct((1, 1), jnp.float32),
        out_specs=pl.BlockSpec(memory_space=pltpu.SMEM),
    )(r(x), r(swu), r(shi), r(shj), r(bs))
    return out[0, 0]


def kernel(u, i, j, W, H, B):
    x, swu, shi, shj, bs = _sc_partials(u, i, j, W, H, B)
    return _tc_reduce(x, swu, shi, shj, bs)


# speedup vs baseline: 1.1419x; 1.1419x over previous
import functools

import jax
import jax.numpy as jnp
from jax import lax
from jax.experimental import pallas as pl
from jax.experimental.pallas import tpu as pltpu
from jax.experimental.pallas import tpu_sc as plsc

N = 16384
DIM = 32
ROWS = 1000000
PACK = 128 // DIM
QROWS = ROWS // PACK
REG_USER = 0.0025
REG_POS_ITEM = 0.0025
REG_NEG_ITEM = 0.00025
REG_BIAS = 0.001

_INFO = plsc.get_sparse_core_info()
_NC = _INFO.num_cores
_NS = _INFO.num_subcores
_NW = _NC * _NS
_L = 16

_CH = 512
_NFULL = (ROWS - 64) // _CH
_TAIL0 = _NFULL * _CH

_BPW = N // _NW
_CHUNK = 256
_NCHUNK = _BPW // _CHUNK


_TCOLS = 2048
_TSUB = _TCOLS // PACK
_TGRID = (ROWS + _TCOLS - 1) // _TCOLS
_QOUT = _TGRID * _TSUB


def _pack_block(x, eye):
    y = jax.lax.dot_general(
        x, eye, (((0,), (0,)), ((), ())),
        preferred_element_type=jnp.float32,
        precision=jax.lax.Precision.HIGHEST,
    )
    return jnp.concatenate(
        [y[p * _TSUB:(p + 1) * _TSUB, :] for p in range(PACK)], axis=1
    )


def _reformat_body(wt_ref, ht_ref, wo_ref, ho_ref):
    eye = jnp.eye(DIM, dtype=jnp.float32)
    wo_ref[...] = _pack_block(wt_ref[...], eye)
    ho_ref[...] = _pack_block(ht_ref[...], eye)


def _k1_body(u_hbm, i_hbm, j_hbm, w_hbm, h_hbm, b_hbm,
             x_hbm, swu_hbm, shi_hbm, shj_hbm, bs_hbm,
             u_v, i_v, j_v, uq_v, iq_v, jq_v, wu_v, hi_v, hj_v, bi_v, bj_v,
             x_v, swu_v, shi_v, shj_v, bs_v, sem):
    wid = lax.axis_index("s") * _NC + lax.axis_index("c")
    base = wid * _BPW

    pltpu.sync_copy(u_hbm.at[pl.ds(base, _BPW)], u_v)
    pltpu.sync_copy(i_hbm.at[pl.ds(base, _BPW)], i_v)
    pltpu.sync_copy(j_hbm.at[pl.ds(base, _BPW)], j_v)

    def srow(v):
        return jnp.bitwise_or(
            jnp.left_shift(jnp.right_shift(v, 11), 9),
            jnp.bitwise_and(v, _TSUB - 1),
        )

    for s in range(_BPW // _L):
        sl = pl.ds(s * _L, _L)
        uq_v[sl] = srow(u_v[sl])
        iq_v[sl] = srow(i_v[sl])
        jq_v[sl] = srow(j_v[sl])

    cb1 = pltpu.async_copy(b_hbm.at[i_v], bi_v, sem)
    cb2 = pltpu.async_copy(b_hbm.at[j_v], bj_v, sem)

    iota = lax.iota(jnp.int32, _L)

    for ch in range(_NCHUNK):
        c0 = ch * _CHUNK
        c1 = pltpu.async_copy(w_hbm.at[uq_v.at[pl.ds(c0, _CHUNK)]], wu_v, sem)
        c2 = pltpu.async_copy(h_hbm.at[iq_v.at[pl.ds(c0, _CHUNK)]], hi_v, sem)
        c3 = pltpu.async_copy(h_hbm.at[jq_v.at[pl.ds(c0, _CHUNK)]], hj_v, sem)
        c1.wait()
        c2.wait()
        c3.wait()

        def block(blk, _):
            r0 = blk * _L
            rows = r0 + iota
            g0 = c0 + r0

            def soff(v):
                return jnp.left_shift(
                    jnp.bitwise_and(jnp.right_shift(v, 9), PACK - 1), 5
                )

            uoff = soff(u_v[pl.ds(g0, _L)])
            ioff = soff(i_v[pl.ds(g0, _L)])
            joff = soff(j_v[pl.ds(g0, _L)])
            zero = jnp.zeros((_L,), jnp.float32)
            acc_ui = zero
            acc_uj = zero
            acc_wu = zero
            acc_hi = zero
            acc_hj = zero
            for d in range(DIM):
                cw = plsc.load_gather(wu_v, [rows, uoff + d])
                ci = plsc.load_gather(hi_v, [rows, ioff + d])
                cj = plsc.load_gather(hj_v, [rows, joff + d])
                acc_ui = acc_ui + cw * ci
                acc_uj = acc_uj + cw * cj
                acc_wu = acc_wu + cw * cw
                acc_hi = acc_hi + ci * ci
                acc_hj = acc_hj + cj * cj
            bi = bi_v[pl.ds(g0, _L)]
            bj = bj_v[pl.ds(g0, _L)]
            x_v[pl.ds(g0, _L)] = acc_ui - acc_uj + bi - bj
            swu_v[pl.ds(g0, _L)] = acc_wu
            shi_v[pl.ds(g0, _L)] = acc_hi
            shj_v[pl.ds(g0, _L)] = acc_hj
            bs_v[pl.ds(g0, _L)] = bi + bj
            return 0

        if ch == 0:
            cb1.wait()
            cb2.wait()
        lax.fori_loop(0, _CHUNK // _L, block, 0)

    pltpu.sync_copy(x_v, x_hbm.at[pl.ds(base, _BPW)])
    pltpu.sync_copy(swu_v, swu_hbm.at[pl.ds(base, _BPW)])
    pltpu.sync_copy(shi_v, shi_hbm.at[pl.ds(base, _BPW)])
    pltpu.sync_copy(shj_v, shj_hbm.at[pl.ds(base, _BPW)])
    pltpu.sync_copy(bs_v, bs_hbm.at[pl.ds(base, _BPW)])


@jax.jit
def _sc_partials(u, i, j, W, H, B):
    f32 = jnp.float32
    mesh = plsc.VectorSubcoreMesh(core_axis_name="c", subcore_axis_name="s")
    params = pltpu.CompilerParams(needs_layout_passes=False)

    spec_in = pl.BlockSpec((DIM, _TCOLS), lambda g: (0, g))
    spec_out = pl.BlockSpec((_TSUB, PACK * DIM), lambda g: (g, 0))
    Ww, Hw = pl.pallas_call(
        _reformat_body,
        grid=(_TGRID,),
        in_specs=[spec_in, spec_in],
        out_specs=[spec_out, spec_out],
        out_shape=[jax.ShapeDtypeStruct((_QOUT, PACK * DIM), f32)] * 2,
    )(W.T, H.T)

    out = pl.kernel(
        _k1_body,
        mesh=mesh,
        compiler_params=params,
        out_type=[jax.ShapeDtypeStruct((N,), f32) for _ in range(5)],
        scratch_types=[
            pltpu.VMEM((_BPW,), jnp.int32),
            pltpu.VMEM((_BPW,), jnp.int32),
            pltpu.VMEM((_BPW,), jnp.int32),
            pltpu.VMEM((_BPW,), jnp.int32),
            pltpu.VMEM((_BPW,), jnp.int32),
            pltpu.VMEM((_BPW,), jnp.int32),
            pltpu.VMEM((_CHUNK, PACK * DIM), f32),
            pltpu.VMEM((_CHUNK, PACK * DIM), f32),
            pltpu.VMEM((_CHUNK, PACK * DIM), f32),
            pltpu.VMEM((_BPW,), f32),
            pltpu.VMEM((_BPW,), f32),
            pltpu.VMEM((_BPW,), f32),
            pltpu.VMEM((_BPW,), f32),
            pltpu.VMEM((_BPW,), f32),
            pltpu.VMEM((_BPW,), f32),
            pltpu.VMEM((_BPW,), f32),
            pltpu.SemaphoreType.DMA,
        ],
    )(u, i, j, Ww, Hw, B)
    return out


def _tc_body(x_ref, swu_ref, shi_ref, shj_ref, bs_ref, out_ref):
    x = x_ref[...]
    lp = jnp.mean(-jnp.log(1.0 + jnp.exp(-x)))
    lp = lp - REG_USER * jnp.mean(jnp.sqrt(swu_ref[...]))
    lp = lp - REG_POS_ITEM * jnp.mean(jnp.sqrt(shi_ref[...]))
    lp = lp - REG_NEG_ITEM * jnp.mean(jnp.sqrt(shj_ref[...]))
    lp = lp - REG_BIAS * jnp.mean(bs_ref[...])
    out_ref[0, 0] = -lp


@jax.jit
def _tc_reduce(x, swu, shi, shj, bs):
    r = lambda a: a.reshape(128, 128)
    out = pl.pallas_call(
        _tc_body,
        out_shape=jax.ShapeDtypeStruct((1, 1), jnp.float32),
        out_specs=pl.BlockSpec(memory_space=pltpu.SMEM),
    )(r(x), r(swu), r(shi), r(shj), r(bs))
    return out[0, 0]


def kernel(u, i, j, W, H, B):
    x, swu, shi, shj, bs = _sc_partials(u, i, j, W, H, B)
    return _tc_reduce(x, swu, shi, shj, bs)


# speedup vs baseline: 2.2120x; 1.9372x over previous
import functools

import jax
import jax.numpy as jnp
from jax import lax
from jax.experimental import pallas as pl
from jax.experimental.pallas import tpu as pltpu
from jax.experimental.pallas import tpu_sc as plsc

N = 16384
DIM = 32
ROWS = 1000000
PACK = 128 // DIM
QROWS = ROWS // PACK
REG_USER = 0.0025
REG_POS_ITEM = 0.0025
REG_NEG_ITEM = 0.00025
REG_BIAS = 0.001

_INFO = plsc.get_sparse_core_info()
_NC = _INFO.num_cores
_NS = _INFO.num_subcores
_NW = _NC * _NS
_L = 16

_CH = 512
_NFULL = (ROWS - 64) // _CH
_TAIL0 = _NFULL * _CH

_BPW = N // _NW
_CHUNK = 256
_NCHUNK = _BPW // _CHUNK


_CH = 512
_NFULL = (ROWS - 64) // _CH
_TAIL0 = _NFULL * _CH
_QCH = _CH // PACK


def _repack_chunk(cbuf, obuf, iota, width):
    def g_body(g, _):
        r16 = g * _L + iota
        r32 = r16 * DIM
        for d in range(DIM):
            t = r32 + jnp.bitwise_and(d + r16, DIM - 1)
            plsc.store_scatter(
                obuf,
                [jnp.right_shift(t, 7), jnp.bitwise_and(t, 127)],
                cbuf[d, pl.ds(g * _L, _L)],
            )
        return 0

    lax.fori_loop(0, width // _L, g_body, 0)


def _k0_body(w_hbm, h_hbm, wo_hbm, ho_hbm,
             cw_v, ch_v, ow_v, oh_v, ct_v, isem, osem):
    wid = lax.axis_index("s") * _NC + lax.axis_index("c")
    lo = jnp.right_shift(wid * _NFULL, 5)
    hi = jnp.right_shift((wid + 1) * _NFULL, 5)
    iota = lax.iota(jnp.int32, _L)

    pltpu.async_copy(w_hbm.at[:, pl.ds(lo * _CH, _CH)], cw_v, isem)
    pltpu.async_copy(h_hbm.at[:, pl.ds(lo * _CH, _CH)], ch_v, isem)

    def chunk(c, _):
        q0 = c * _QCH

        @pl.when(c > lo)
        def _drain_outs():
            pltpu.make_async_copy(ow_v, wo_hbm.at[pl.ds(0, _QCH)], osem).wait()
            pltpu.make_async_copy(oh_v, ho_hbm.at[pl.ds(0, _QCH)], osem).wait()

        pltpu.make_async_copy(w_hbm.at[:, pl.ds(0, _CH)], cw_v, isem).wait()
        pltpu.make_async_copy(w_hbm.at[:, pl.ds(0, _CH)], ch_v, isem).wait()
        _repack_chunk(cw_v, ow_v, iota, _CH)
        pltpu.async_copy(ow_v, wo_hbm.at[pl.ds(q0, _QCH)], osem)
        _repack_chunk(ch_v, oh_v, iota, _CH)
        pltpu.async_copy(oh_v, ho_hbm.at[pl.ds(q0, _QCH)], osem)

        @pl.when(c + 1 < hi)
        def _prefetch():
            c1 = (c + 1) * _CH
            pltpu.async_copy(w_hbm.at[:, pl.ds(c1, _CH)], cw_v, isem)
            pltpu.async_copy(h_hbm.at[:, pl.ds(c1, _CH)], ch_v, isem)

        return 0

    lax.fori_loop(lo, hi, chunk, 0)
    pltpu.make_async_copy(ow_v, wo_hbm.at[pl.ds(0, _QCH)], osem).wait()
    pltpu.make_async_copy(oh_v, ho_hbm.at[pl.ds(0, _QCH)], osem).wait()

    @pl.when(wid == 0)
    def _tail():
        tw = 64
        qt = _TAIL0 // PACK
        pltpu.async_copy(w_hbm.at[:, pl.ds(_TAIL0, tw)], ct_v, isem).wait()
        _repack_chunk(ct_v, ow_v, iota, tw)
        pltpu.sync_copy(ow_v.at[pl.ds(0, tw // PACK)], wo_hbm.at[pl.ds(qt, tw // PACK)])
        pltpu.async_copy(h_hbm.at[:, pl.ds(_TAIL0, tw)], ct_v, isem).wait()
        _repack_chunk(ct_v, oh_v, iota, tw)
        pltpu.sync_copy(oh_v.at[pl.ds(0, tw // PACK)], ho_hbm.at[pl.ds(qt, tw // PACK)])


def _k1_body(u_hbm, i_hbm, j_hbm, w_hbm, h_hbm, b_hbm,
             x_hbm, swu_hbm, shi_hbm, shj_hbm, bs_hbm,
             u_v, i_v, j_v, uq_v, iq_v, jq_v, wu_v, hi_v, hj_v, bi_v, bj_v,
             x_v, swu_v, shi_v, shj_v, bs_v, sem):
    wid = lax.axis_index("s") * _NC + lax.axis_index("c")
    base = wid * _BPW

    pltpu.sync_copy(u_hbm.at[pl.ds(base, _BPW)], u_v)
    pltpu.sync_copy(i_hbm.at[pl.ds(base, _BPW)], i_v)
    pltpu.sync_copy(j_hbm.at[pl.ds(base, _BPW)], j_v)

    for s in range(_BPW // _L):
        sl = pl.ds(s * _L, _L)
        uq_v[sl] = jnp.right_shift(u_v[sl], 2)
        iq_v[sl] = jnp.right_shift(i_v[sl], 2)
        jq_v[sl] = jnp.right_shift(j_v[sl], 2)

    cb1 = pltpu.async_copy(b_hbm.at[i_v], bi_v, sem)
    cb2 = pltpu.async_copy(b_hbm.at[j_v], bj_v, sem)

    iota = lax.iota(jnp.int32, _L)

    for ch in range(_NCHUNK):
        c0 = ch * _CHUNK
        c1 = pltpu.async_copy(w_hbm.at[uq_v.at[pl.ds(c0, _CHUNK)]], wu_v, sem)
        c2 = pltpu.async_copy(h_hbm.at[iq_v.at[pl.ds(c0, _CHUNK)]], hi_v, sem)
        c3 = pltpu.async_copy(h_hbm.at[jq_v.at[pl.ds(c0, _CHUNK)]], hj_v, sem)
        c1.wait()
        c2.wait()
        c3.wait()

        def block(blk, _):
            r0 = blk * _L
            rows = r0 + iota
            g0 = c0 + r0

            uu = u_v[pl.ds(g0, _L)]
            ii = i_v[pl.ds(g0, _L)]
            jj = j_v[pl.ds(g0, _L)]

            def soff(v):
                return jnp.left_shift(jnp.bitwise_and(v, PACK - 1), 5)

            uoff = soff(uu)
            ioff = soff(ii)
            joff = soff(jj)
            urot = jnp.bitwise_and(uu, DIM - 1)
            irot = jnp.bitwise_and(ii, DIM - 1)
            jrot = jnp.bitwise_and(jj, DIM - 1)
            zero = jnp.zeros((_L,), jnp.float32)
            acc_ui = zero
            acc_uj = zero
            acc_wu = zero
            acc_hi = zero
            acc_hj = zero
            for d in range(DIM):
                cw = plsc.load_gather(
                    wu_v, [rows, uoff + jnp.bitwise_and(urot + d, DIM - 1)])
                ci = plsc.load_gather(
                    hi_v, [rows, ioff + jnp.bitwise_and(irot + d, DIM - 1)])
                cj = plsc.load_gather(
                    hj_v, [rows, joff + jnp.bitwise_and(jrot + d, DIM - 1)])
                acc_ui = acc_ui + cw * ci
                acc_uj = acc_uj + cw * cj
                acc_wu = acc_wu + cw * cw
                acc_hi = acc_hi + ci * ci
                acc_hj = acc_hj + cj * cj
            bi = bi_v[pl.ds(g0, _L)]
            bj = bj_v[pl.ds(g0, _L)]
            x_v[pl.ds(g0, _L)] = acc_ui - acc_uj + bi - bj
            swu_v[pl.ds(g0, _L)] = acc_wu
            shi_v[pl.ds(g0, _L)] = acc_hi
            shj_v[pl.ds(g0, _L)] = acc_hj
            bs_v[pl.ds(g0, _L)] = bi + bj
            return 0

        if ch == 0:
            cb1.wait()
            cb2.wait()
        lax.fori_loop(0, _CHUNK // _L, block, 0)

    pltpu.sync_copy(x_v, x_hbm.at[pl.ds(base, _BPW)])
    pltpu.sync_copy(swu_v, swu_hbm.at[pl.ds(base, _BPW)])
    pltpu.sync_copy(shi_v, shi_hbm.at[pl.ds(base, _BPW)])
    pltpu.sync_copy(shj_v, shj_hbm.at[pl.ds(base, _BPW)])
    pltpu.sync_copy(bs_v, bs_hbm.at[pl.ds(base, _BPW)])


@jax.jit
def _sc_partials(u, i, j, W, H, B):
    f32 = jnp.float32
    mesh = plsc.VectorSubcoreMesh(core_axis_name="c", subcore_axis_name="s")
    params = pltpu.CompilerParams(needs_layout_passes=False)

    Ww, Hw = pl.kernel(
        _k0_body,
        mesh=mesh,
        compiler_params=params,
        out_type=[jax.ShapeDtypeStruct((QROWS, PACK * DIM), f32)] * 2,
        scratch_types=[
            pltpu.VMEM((DIM, _CH), f32),
            pltpu.VMEM((DIM, _CH), f32),
            pltpu.VMEM((_QCH, PACK * DIM), f32),
            pltpu.VMEM((_QCH, PACK * DIM), f32),
            pltpu.VMEM((DIM, 64), f32),
            pltpu.SemaphoreType.DMA,
            pltpu.SemaphoreType.DMA,
        ],
    )(W.T, H.T)

    out = pl.kernel(
        _k1_body,
        mesh=mesh,
        compiler_params=params,
        out_type=[jax.ShapeDtypeStruct((N,), f32) for _ in range(5)],
        scratch_types=[
            pltpu.VMEM((_BPW,), jnp.int32),
            pltpu.VMEM((_BPW,), jnp.int32),
            pltpu.VMEM((_BPW,), jnp.int32),
            pltpu.VMEM((_BPW,), jnp.int32),
            pltpu.VMEM((_BPW,), jnp.int32),
            pltpu.VMEM((_BPW,), jnp.int32),
            pltpu.VMEM((_CHUNK, PACK * DIM), f32),
            pltpu.VMEM((_CHUNK, PACK * DIM), f32),
            pltpu.VMEM((_CHUNK, PACK * DIM), f32),
            pltpu.VMEM((_BPW,), f32),
            pltpu.VMEM((_BPW,), f32),
            pltpu.VMEM((_BPW,), f32),
            pltpu.VMEM((_BPW,), f32),
            pltpu.VMEM((_BPW,), f32),
            pltpu.VMEM((_BPW,), f32),
            pltpu.VMEM((_BPW,), f32),
            pltpu.SemaphoreType.DMA,
        ],
    )(u, i, j, Ww, Hw, B)
    return out


def _tc_body(x_ref, swu_ref, shi_ref, shj_ref, bs_ref, out_ref):
    x = x_ref[...]
    lp = jnp.mean(-jnp.log(1.0 + jnp.exp(-x)))
    lp = lp - REG_USER * jnp.mean(jnp.sqrt(swu_ref[...]))
    lp = lp - REG_POS_ITEM * jnp.mean(jnp.sqrt(shi_ref[...]))
    lp = lp - REG_NEG_ITEM * jnp.mean(jnp.sqrt(shj_ref[...]))
    lp = lp - REG_BIAS * jnp.mean(bs_ref[...])
    out_ref[0, 0] = -lp


@jax.jit
def _tc_reduce(x, swu, shi, shj, bs):
    r = lambda a: a.reshape(128, 128)
    out = pl.pallas_call(
        _tc_body,
        out_shape=jax.ShapeDtypeStruct((1, 1), jnp.float32),
        out_specs=pl.BlockSpec(memory_space=pltpu.SMEM),
    )(r(x), r(swu), r(shi), r(shj), r(bs))
    return out[0, 0]


def kernel(u, i, j, W, H, B):
    x, swu, shi, shj, bs = _sc_partials(u, i, j, W, H, B)
    return _tc_reduce(x, swu, shi, shj, bs)


# speedup vs baseline: 2.2283x; 1.0073x over previous
import functools

import jax
import jax.numpy as jnp
from jax import lax
from jax.experimental import pallas as pl
from jax.experimental.pallas import tpu as pltpu
from jax.experimental.pallas import tpu_sc as plsc

N = 16384
DIM = 32
ROWS = 1000000
PACK = 128 // DIM
QROWS = ROWS // PACK
REG_USER = 0.0025
REG_POS_ITEM = 0.0025
REG_NEG_ITEM = 0.00025
REG_BIAS = 0.001

_INFO = plsc.get_sparse_core_info()
_NC = _INFO.num_cores
_NS = _INFO.num_subcores
_NW = _NC * _NS
_L = 16

_CH = 512
_NFULL = (ROWS - 64) // _CH
_TAIL0 = _NFULL * _CH

_BPW = N // _NW
_CHUNK = 256
_NCHUNK = _BPW // _CHUNK


_CH = 512
_NFULL = (ROWS - 64) // _CH
_TAIL0 = _NFULL * _CH
_QCH = _CH // PACK


def _repack_chunk(cbuf, obuf, iota, width):
    def g_body(g, _):
        r16 = g * _L + iota
        row = jnp.right_shift(r16, 2)
        colbase = jnp.left_shift(jnp.bitwise_and(r16, 3), 5)
        for d in range(DIM):
            col = colbase + jnp.bitwise_and(d + iota, DIM - 1)
            plsc.store_scatter(obuf, [row, col], cbuf[d, pl.ds(g * _L, _L)])
        return 0

    lax.fori_loop(0, width // _L, g_body, 0)


def _k0_body(w_hbm, h_hbm, wo_hbm, ho_hbm,
             cw_v, ch_v, ow_v, oh_v, ct_v, isem, osem):
    wid = lax.axis_index("s") * _NC + lax.axis_index("c")
    lo = jnp.right_shift(wid * _NFULL, 5)
    hi = jnp.right_shift((wid + 1) * _NFULL, 5)
    iota = lax.iota(jnp.int32, _L)

    pltpu.async_copy(w_hbm.at[:, pl.ds(lo * _CH, _CH)], cw_v, isem)
    pltpu.async_copy(h_hbm.at[:, pl.ds(lo * _CH, _CH)], ch_v, isem)

    def chunk(c, _):
        q0 = c * _QCH

        @pl.when(c > lo)
        def _drain_outs():
            pltpu.make_async_copy(ow_v, wo_hbm.at[pl.ds(0, _QCH)], osem).wait()
            pltpu.make_async_copy(oh_v, ho_hbm.at[pl.ds(0, _QCH)], osem).wait()

        pltpu.make_async_copy(w_hbm.at[:, pl.ds(0, _CH)], cw_v, isem).wait()
        pltpu.make_async_copy(w_hbm.at[:, pl.ds(0, _CH)], ch_v, isem).wait()
        _repack_chunk(cw_v, ow_v, iota, _CH)
        pltpu.async_copy(ow_v, wo_hbm.at[pl.ds(q0, _QCH)], osem)
        _repack_chunk(ch_v, oh_v, iota, _CH)
        pltpu.async_copy(oh_v, ho_hbm.at[pl.ds(q0, _QCH)], osem)

        @pl.when(c + 1 < hi)
        def _prefetch():
            c1 = (c + 1) * _CH
            pltpu.async_copy(w_hbm.at[:, pl.ds(c1, _CH)], cw_v, isem)
            pltpu.async_copy(h_hbm.at[:, pl.ds(c1, _CH)], ch_v, isem)

        return 0

    lax.fori_loop(lo, hi, chunk, 0)
    pltpu.make_async_copy(ow_v, wo_hbm.at[pl.ds(0, _QCH)], osem).wait()
    pltpu.make_async_copy(oh_v, ho_hbm.at[pl.ds(0, _QCH)], osem).wait()

    @pl.when(wid == 0)
    def _tail():
        tw = 64
        qt = _TAIL0 // PACK
        pltpu.async_copy(w_hbm.at[:, pl.ds(_TAIL0, tw)], ct_v, isem).wait()
        _repack_chunk(ct_v, ow_v, iota, tw)
        pltpu.sync_copy(ow_v.at[pl.ds(0, tw // PACK)], wo_hbm.at[pl.ds(qt, tw // PACK)])
        pltpu.async_copy(h_hbm.at[:, pl.ds(_TAIL0, tw)], ct_v, isem).wait()
        _repack_chunk(ct_v, oh_v, iota, tw)
        pltpu.sync_copy(oh_v.at[pl.ds(0, tw // PACK)], ho_hbm.at[pl.ds(qt, tw // PACK)])


def _k1_body(u_hbm, i_hbm, j_hbm, w_hbm, h_hbm, b_hbm,
             x_hbm, swu_hbm, shi_hbm, shj_hbm, bs_hbm,
             u_v, i_v, j_v, uq_v, iq_v, jq_v, wu_v, hi_v, hj_v, bi_v, bj_v,
             x_v, swu_v, shi_v, shj_v, bs_v, sem):
    wid = lax.axis_index("s") * _NC + lax.axis_index("c")
    base = wid * _BPW

    pltpu.sync_copy(u_hbm.at[pl.ds(base, _BPW)], u_v)
    pltpu.sync_copy(i_hbm.at[pl.ds(base, _BPW)], i_v)
    pltpu.sync_copy(j_hbm.at[pl.ds(base, _BPW)], j_v)

    for s in range(_BPW // _L):
        sl = pl.ds(s * _L, _L)
        uq_v[sl] = jnp.right_shift(u_v[sl], 2)
        iq_v[sl] = jnp.right_shift(i_v[sl], 2)
        jq_v[sl] = jnp.right_shift(j_v[sl], 2)

    cb1 = pltpu.async_copy(b_hbm.at[i_v], bi_v, sem)
    cb2 = pltpu.async_copy(b_hbm.at[j_v], bj_v, sem)

    iota = lax.iota(jnp.int32, _L)

    for ch in range(_NCHUNK):
        c0 = ch * _CHUNK
        c1 = pltpu.async_copy(w_hbm.at[uq_v.at[pl.ds(c0, _CHUNK)]], wu_v, sem)
        c2 = pltpu.async_copy(h_hbm.at[iq_v.at[pl.ds(c0, _CHUNK)]], hi_v, sem)
        c3 = pltpu.async_copy(h_hbm.at[jq_v.at[pl.ds(c0, _CHUNK)]], hj_v, sem)
        c1.wait()
        c2.wait()
        c3.wait()

        def block(blk, _):
            r0 = blk * _L
            rows = r0 + iota
            g0 = c0 + r0

            uu = u_v[pl.ds(g0, _L)]
            ii = i_v[pl.ds(g0, _L)]
            jj = j_v[pl.ds(g0, _L)]

            def soff(v):
                return jnp.left_shift(jnp.bitwise_and(v, PACK - 1), 5)

            uoff = soff(uu)
            ioff = soff(ii)
            joff = soff(jj)
            urot = jnp.bitwise_and(uu, _L - 1)
            irot = jnp.bitwise_and(ii, _L - 1)
            jrot = jnp.bitwise_and(jj, _L - 1)
            zero = jnp.zeros((_L,), jnp.float32)
            acc_ui = zero
            acc_uj = zero
            acc_wu = zero
            acc_hi = zero
            acc_hj = zero
            for d in range(DIM):
                cw = plsc.load_gather(
                    wu_v, [rows, uoff + jnp.bitwise_and(urot + d, DIM - 1)])
                ci = plsc.load_gather(
                    hi_v, [rows, ioff + jnp.bitwise_and(irot + d, DIM - 1)])
                cj = plsc.load_gather(
                    hj_v, [rows, joff + jnp.bitwise_and(jrot + d, DIM - 1)])
                acc_ui = acc_ui + cw * ci
                acc_uj = acc_uj + cw * cj
                acc_wu = acc_wu + cw * cw
                acc_hi = acc_hi + ci * ci
                acc_hj = acc_hj + cj * cj
            bi = bi_v[pl.ds(g0, _L)]
            bj = bj_v[pl.ds(g0, _L)]
            x_v[pl.ds(g0, _L)] = acc_ui - acc_uj + bi - bj
            swu_v[pl.ds(g0, _L)] = acc_wu
            shi_v[pl.ds(g0, _L)] = acc_hi
            shj_v[pl.ds(g0, _L)] = acc_hj
            bs_v[pl.ds(g0, _L)] = bi + bj
            return 0

        if ch == 0:
            cb1.wait()
            cb2.wait()
        lax.fori_loop(0, _CHUNK // _L, block, 0)

    pltpu.sync_copy(x_v, x_hbm.at[pl.ds(base, _BPW)])
    pltpu.sync_copy(swu_v, swu_hbm.at[pl.ds(base, _BPW)])
    pltpu.sync_copy(shi_v, shi_hbm.at[pl.ds(base, _BPW)])
    pltpu.sync_copy(shj_v, shj_hbm.at[pl.ds(base, _BPW)])
    pltpu.sync_copy(bs_v, bs_hbm.at[pl.ds(base, _BPW)])


@jax.jit
def _sc_partials(u, i, j, W, H, B):
    f32 = jnp.float32
    mesh = plsc.VectorSubcoreMesh(core_axis_name="c", subcore_axis_name="s")
    params = pltpu.CompilerParams(needs_layout_passes=False)

    Ww, Hw = pl.kernel(
        _k0_body,
        mesh=mesh,
        compiler_params=params,
        out_type=[jax.ShapeDtypeStruct((QROWS, PACK * DIM), f32)] * 2,
        scratch_types=[
            pltpu.VMEM((DIM, _CH), f32),
            pltpu.VMEM((DIM, _CH), f32),
            pltpu.VMEM((_QCH, PACK * DIM), f32),
            pltpu.VMEM((_QCH, PACK * DIM), f32),
            pltpu.VMEM((DIM, 64), f32),
            pltpu.SemaphoreType.DMA,
            pltpu.SemaphoreType.DMA,
        ],
    )(W.T, H.T)

    out = pl.kernel(
        _k1_body,
        mesh=mesh,
        compiler_params=params,
        out_type=[jax.ShapeDtypeStruct((N,), f32) for _ in range(5)],
        scratch_types=[
            pltpu.VMEM((_BPW,), jnp.int32),
            pltpu.VMEM((_BPW,), jnp.int32),
            pltpu.VMEM((_BPW,), jnp.int32),
            pltpu.VMEM((_BPW,), jnp.int32),
            pltpu.VMEM((_BPW,), jnp.int32),
            pltpu.VMEM((_BPW,), jnp.int32),
            pltpu.VMEM((_CHUNK, PACK * DIM), f32),
            pltpu.VMEM((_CHUNK, PACK * DIM), f32),
            pltpu.VMEM((_CHUNK, PACK * DIM), f32),
            pltpu.VMEM((_BPW,), f32),
            pltpu.VMEM((_BPW,), f32),
            pltpu.VMEM((_BPW,), f32),
            pltpu.VMEM((_BPW,), f32),
            pltpu.VMEM((_BPW,), f32),
            pltpu.VMEM((_BPW,), f32),
            pltpu.VMEM((_BPW,), f32),
            pltpu.SemaphoreType.DMA,
        ],
    )(u, i, j, Ww, Hw, B)
    return out


def _tc_body(x_ref, swu_ref, shi_ref, shj_ref, bs_ref, out_ref):
    x = x_ref[...]
    lp = jnp.mean(-jnp.log(1.0 + jnp.exp(-x)))
    lp = lp - REG_USER * jnp.mean(jnp.sqrt(swu_ref[...]))
    lp = lp - REG_POS_ITEM * jnp.mean(jnp.sqrt(shi_ref[...]))
    lp = lp - REG_NEG_ITEM * jnp.mean(jnp.sqrt(shj_ref[...]))
    lp = lp - REG_BIAS * jnp.mean(bs_ref[...])
    out_ref[0, 0] = -lp


@jax.jit
def _tc_reduce(x, swu, shi, shj, bs):
    r = lambda a: a.reshape(128, 128)
    out = pl.pallas_call(
        _tc_body,
        out_shape=jax.ShapeDtypeStruct((1, 1), jnp.float32),
        out_specs=pl.BlockSpec(memory_space=pltpu.SMEM),
    )(r(x), r(swu), r(shi), r(shj), r(bs))
    return out[0, 0]


def kernel(u, i, j, W, H, B):
    x, swu, shi, shj, bs = _sc_partials(u, i, j, W, H, B)
    return _tc_reduce(x, swu, shi, shj, bs)
